# Initial kernel scaffold; baseline (speedup 1.0000x reference)
import jax
import jax.numpy as jnp
from jax.experimental import pallas as pl


def kernel(predict, target, penalty_matrix):
    raise NotImplementedError("write your pallas kernel here")


# speedup vs baseline: 1.3563x; 1.3563x over previous
import jax
import jax.numpy as jnp
from jax import lax
from jax.experimental import pallas as pl
from jax.experimental.pallas import tpu as pltpu
from jax.experimental.pallas import tpu_sc as plsc

B = 16384
W = 100
NC = 2
NS = 16
L = 16
NW = NC * NS
RPT = B // NW
GROUPS = RPT // L


def _sc_body(p_hbm, t_hbm, pm_hbm, out_hbm, pbuf, tbuf, pmbuf, obuf):
    wid = lax.axis_index("s") * NC + lax.axis_index("c")
    base = wid * (RPT * W)
    pltpu.sync_copy(pm_hbm, pmbuf)
    pltpu.sync_copy(p_hbm.at[pl.ds(base, RPT * W)], pbuf)
    pltpu.sync_copy(t_hbm.at[pl.ds(base, RPT * W)], tbuf)

    lane = lax.broadcasted_iota(jnp.int32, (L,), 0)
    zero_i = jnp.zeros((L,), jnp.int32)
    zero_f = jnp.zeros((L,), jnp.float32)
    neg = jnp.full((L,), -3.0e38, jnp.float32)

    def group_body(g, acc):
        rowbase = g * (L * W) + lane * W

        def col_body(c, carry):
            vmax, vidx, tmax, tidx, esum = carry
            idx = rowbase + c
            v = plsc.load_gather(pbuf, [idx])
            t = plsc.load_gather(tbuf, [idx])
            esum = esum + jnp.exp(v)
            colv = zero_i + c
            bp = v > vmax
            vmax = jnp.where(bp, v, vmax)
            vidx = jnp.where(bp, colv, vidx)
            bt = t > tmax
            tmax = jnp.where(bt, t, tmax)
            tidx = jnp.where(bt, colv, tidx)
            return (vmax, vidx, tmax, tidx, esum)

        vmax, vidx, tmax, tidx, esum = lax.fori_loop(
            0, W, col_body, (neg, zero_i, neg, zero_i, zero_f))
        jidx = tidx * W + vidx
        pmv = plsc.load_gather(pmbuf, [jidx])
        contrib = jnp.where(vidx != tidx, pmv * jnp.exp(vmax) / esum, zero_f)
        return acc + contrib

    acc = lax.fori_loop(0, GROUPS, group_body, zero_f)
    obuf[...] = acc
    pltpu.sync_copy(obuf, out_hbm.at[wid])


def kernel(predict, target, penalty_matrix):
    mesh = plsc.VectorSubcoreMesh(core_axis_name="c", subcore_axis_name="s")
    partials = pl.kernel(
        _sc_body,
        out_type=jax.ShapeDtypeStruct((NW, L), jnp.float32),
        mesh=mesh,
        scratch_types=[
            pltpu.VMEM((RPT * W,), jnp.float32),
            pltpu.VMEM((RPT * W,), jnp.float32),
            pltpu.VMEM((W * W,), jnp.float32),
            pltpu.VMEM((L,), jnp.float32),
        ],
        compiler_params=pltpu.CompilerParams(needs_layout_passes=False),
    )(predict.reshape(-1), target.reshape(-1), penalty_matrix.reshape(-1))
    return jnp.sum(partials) / B


# speedup vs baseline: 1.4940x; 1.1016x over previous
import jax
import jax.numpy as jnp
from jax import lax
from jax.experimental import pallas as pl
from jax.experimental.pallas import tpu as pltpu
from jax.experimental.pallas import tpu_sc as plsc

B = 16384
W = 100
NC = 2
NS = 16
L = 16
NW = NC * NS
RPT = B // NW
GROUPS = RPT // L


def _sc_body(p_hbm, t_hbm, pm_hbm, out_hbm, pbuf, tbuf, pmbuf, obuf):
    wid = lax.axis_index("s") * NC + lax.axis_index("c")
    base = wid * (RPT * W)
    pltpu.sync_copy(pm_hbm, pmbuf)
    pltpu.sync_copy(p_hbm.at[pl.ds(base, RPT * W)], pbuf)
    pltpu.sync_copy(t_hbm.at[pl.ds(base, RPT * W)], tbuf)

    lane = lax.broadcasted_iota(jnp.int32, (L,), 0)
    zero_i = jnp.zeros((L,), jnp.int32)
    zero_f = jnp.zeros((L,), jnp.float32)
    neg = jnp.full((L,), -3.0e38, jnp.float32)

    U = 4

    def group_body(g, acc):
        rowbase = g * (L * W) + lane * W

        def col_body(i, carry):
            vmax, vidx, tmax, tidx, esum = carry
            c0 = i * U
            for k in range(U):
                c = c0 + k
                idx = rowbase + c
                v = plsc.load_gather(pbuf, [idx])
                t = plsc.load_gather(tbuf, [idx])
                esum = esum + jnp.exp(v)
                colv = zero_i + c
                bp = v > vmax
                vmax = jnp.where(bp, v, vmax)
                vidx = jnp.where(bp, colv, vidx)
                bt = t > tmax
                tmax = jnp.where(bt, t, tmax)
                tidx = jnp.where(bt, colv, tidx)
            return (vmax, vidx, tmax, tidx, esum)

        vmax, vidx, tmax, tidx, esum = lax.fori_loop(
            0, W // U, col_body, (neg, zero_i, neg, zero_i, zero_f))
        jidx = tidx * W + vidx
        pmv = plsc.load_gather(pmbuf, [jidx])
        contrib = jnp.where(vidx != tidx, pmv * jnp.exp(vmax) / esum, zero_f)
        return acc + contrib

    acc = lax.fori_loop(0, GROUPS, group_body, zero_f)
    obuf[...] = acc
    pltpu.sync_copy(obuf, out_hbm.at[wid])


def kernel(predict, target, penalty_matrix):
    mesh = plsc.VectorSubcoreMesh(core_axis_name="c", subcore_axis_name="s")
    partials = pl.kernel(
        _sc_body,
        out_type=jax.ShapeDtypeStruct((NW, L), jnp.float32),
        mesh=mesh,
        scratch_types=[
            pltpu.VMEM((RPT * W,), jnp.float32),
            pltpu.VMEM((RPT * W,), jnp.float32),
            pltpu.VMEM((W * W,), jnp.float32),
            pltpu.VMEM((L,), jnp.float32),
        ],
        compiler_params=pltpu.CompilerParams(needs_layout_passes=False),
    )(predict.reshape(-1), target.reshape(-1), penalty_matrix.reshape(-1))
    return jnp.sum(partials) / B


# speedup vs baseline: 1.8257x; 1.2220x over previous
import jax
import jax.numpy as jnp
from jax import lax
from jax.experimental import pallas as pl
from jax.experimental.pallas import tpu as pltpu
from jax.experimental.pallas import tpu_sc as plsc

B = 16384
W = 100
RB = 1024
GRID = B // RB
OR = RB // 128

NC = 2
NS = 16
L = 16
NW = NC * NS
EPT = B // NW
GROUPS = EPT // L


def _tc_body(p_ref, t_ref, scale_ref, jidx_ref):
    p = p_ref[...]
    t = t_ref[...]
    cols = lax.broadcasted_iota(jnp.int32, (RB, W), 1)
    m = jnp.max(p, axis=1, keepdims=True)
    pre = jnp.min(jnp.where(p == m, cols, W), axis=1)
    tm = jnp.max(t, axis=1, keepdims=True)
    tar = jnp.min(jnp.where(t == tm, cols, W), axis=1)
    sumexp = jnp.sum(jnp.exp(p), axis=1)
    scale = jnp.where(pre != tar, jnp.exp(m[:, 0]) / sumexp, 0.0)
    scale_ref[...] = scale.reshape(OR, 128)
    jidx_ref[...] = (tar * W + pre).reshape(OR, 128)


def _sc_body(scale_hbm, jidx_hbm, pm_hbm, out_hbm, sbuf, jbuf, pmbuf, obuf):
    wid = lax.axis_index("s") * NC + lax.axis_index("c")
    base = wid * EPT
    pltpu.sync_copy(pm_hbm, pmbuf)
    pltpu.sync_copy(scale_hbm.at[pl.ds(base, EPT)], sbuf)
    pltpu.sync_copy(jidx_hbm.at[pl.ds(base, EPT)], jbuf)

    def group_body(g, acc):
        o = g * L
        idxv = jbuf[pl.ds(o, L)]
        scv = sbuf[pl.ds(o, L)]
        pmv = plsc.load_gather(pmbuf, [idxv])
        return acc + pmv * scv

    acc = lax.fori_loop(0, GROUPS, group_body, jnp.zeros((L,), jnp.float32))
    obuf[...] = acc
    pltpu.sync_copy(obuf, out_hbm.at[wid])


def kernel(predict, target, penalty_matrix):
    scale2d, jidx2d = pl.pallas_call(
        _tc_body,
        grid=(GRID,),
        in_specs=[
            pl.BlockSpec((RB, W), lambda i: (i, 0)),
            pl.BlockSpec((RB, W), lambda i: (i, 0)),
        ],
        out_specs=[
            pl.BlockSpec((OR, 128), lambda i: (i, 0)),
            pl.BlockSpec((OR, 128), lambda i: (i, 0)),
        ],
        out_shape=[
            jax.ShapeDtypeStruct((B // 128, 128), jnp.float32),
            jax.ShapeDtypeStruct((B // 128, 128), jnp.int32),
        ],
    )(predict, target)

    mesh = plsc.VectorSubcoreMesh(core_axis_name="c", subcore_axis_name="s")
    partials = pl.kernel(
        _sc_body,
        out_type=jax.ShapeDtypeStruct((NW, L), jnp.float32),
        mesh=mesh,
        scratch_types=[
            pltpu.VMEM((EPT,), jnp.float32),
            pltpu.VMEM((EPT,), jnp.int32),
            pltpu.VMEM((W * W,), jnp.float32),
            pltpu.VMEM((L,), jnp.float32),
        ],
        compiler_params=pltpu.CompilerParams(needs_layout_passes=False),
    )(scale2d.reshape(-1), jidx2d.reshape(-1), penalty_matrix.reshape(-1))
    return jnp.sum(partials) / B


# speedup vs baseline: 1.9025x; 1.0421x over previous
import jax
import jax.numpy as jnp
from jax import lax
from jax.experimental import pallas as pl
from jax.experimental.pallas import tpu as pltpu
from jax.experimental.pallas import tpu_sc as plsc

B = 16384
W = 100
RB = 1024
GRID = B // RB
OR = RB // 128

NC = 2
NS = 16
L = 16
NW = NC * NS
EPT = B // NW
GROUPS = EPT // L


def _tc_body(p_ref, t_ref, scale_ref, jidx_ref):
    p = p_ref[...]
    t = t_ref[...]
    colsf = lax.broadcasted_iota(jnp.int32, (RB, W), 1).astype(jnp.float32)
    m = jnp.max(p, axis=1, keepdims=True)
    pre = jnp.min(jnp.where(p == m, colsf, float(W - 1)), axis=1)
    tm = jnp.max(t, axis=1, keepdims=True)
    tar = jnp.min(jnp.where(t == tm, colsf, float(W - 1)), axis=1)
    ones = jnp.ones((W, 1), jnp.float32)
    sumexp = lax.dot_general(
        jnp.exp(p), ones, (((1,), (0,)), ((), ())),
        preferred_element_type=jnp.float32,
    )[:, 0]
    scale = jnp.where(pre != tar, jnp.exp(m[:, 0]) / sumexp, 0.0)
    scale_ref[...] = scale.reshape(OR, 128)
    jidx_ref[...] = (tar * W + pre).reshape(OR, 128).astype(jnp.int32)


def _sc_body(scale_hbm, jidx_hbm, pm_hbm, out_hbm, sbuf, jbuf, pmbuf, obuf):
    wid = lax.axis_index("s") * NC + lax.axis_index("c")
    base = wid * EPT
    pltpu.sync_copy(pm_hbm, pmbuf)
    pltpu.sync_copy(scale_hbm.at[pl.ds(base, EPT)], sbuf)
    pltpu.sync_copy(jidx_hbm.at[pl.ds(base, EPT)], jbuf)

    def group_body(g, acc):
        o = g * L
        idxv = jbuf[pl.ds(o, L)]
        scv = sbuf[pl.ds(o, L)]
        pmv = plsc.load_gather(pmbuf, [idxv])
        return acc + pmv * scv

    acc = lax.fori_loop(0, GROUPS, group_body, jnp.zeros((L,), jnp.float32))
    obuf[...] = acc
    pltpu.sync_copy(obuf, out_hbm.at[wid])


def kernel(predict, target, penalty_matrix):
    scale2d, jidx2d = pl.pallas_call(
        _tc_body,
        grid=(GRID,),
        in_specs=[
            pl.BlockSpec((RB, W), lambda i: (i, 0)),
            pl.BlockSpec((RB, W), lambda i: (i, 0)),
        ],
        out_specs=[
            pl.BlockSpec((OR, 128), lambda i: (i, 0)),
            pl.BlockSpec((OR, 128), lambda i: (i, 0)),
        ],
        out_shape=[
            jax.ShapeDtypeStruct((B // 128, 128), jnp.float32),
            jax.ShapeDtypeStruct((B // 128, 128), jnp.int32),
        ],
    )(predict, target)

    mesh = plsc.VectorSubcoreMesh(core_axis_name="c", subcore_axis_name="s")
    partials = pl.kernel(
        _sc_body,
        out_type=jax.ShapeDtypeStruct((NW, L), jnp.float32),
        mesh=mesh,
        scratch_types=[
            pltpu.VMEM((EPT,), jnp.float32),
            pltpu.VMEM((EPT,), jnp.int32),
            pltpu.VMEM((W * W,), jnp.float32),
            pltpu.VMEM((L,), jnp.float32),
        ],
        compiler_params=pltpu.CompilerParams(needs_layout_passes=False),
    )(scale2d.reshape(-1), jidx2d.reshape(-1), penalty_matrix.reshape(-1))
    return jnp.sum(partials) / B
